# baseline (speedup 1.0000x reference)
import functools

import jax
import jax.numpy as jnp
from jax import lax
from jax.experimental import pallas as pl
from jax.experimental.pallas import tpu as pltpu
from jax.experimental.pallas import tpu_sc as plsc

H = 128
E = 65536
T = 262144
SCN = 8

NC = 2
NS = 16
NW = NC * NS


def _silu(v):
    return v / (1.0 + jnp.exp(-v))


BE = 2048


def _edge_prep(x, rbf, W_rbf, W_ji, b_ji, W_kj, b_kj, S_cat):
    def body(x_ref, rbf_ref, wr_ref, wji_ref, bji_ref, wkj_ref, bkj_ref,
             sc_ref, xji_ref, z_ref):
        xb = x_ref[...]
        rh = jnp.dot(rbf_ref[...], wr_ref[...],
                     preferred_element_type=jnp.float32)
        xji_ref[...] = _silu(jnp.dot(xb, wji_ref[...],
                                     preferred_element_type=jnp.float32)
                             + bji_ref[...])
        xkj = _silu(jnp.dot(xb, wkj_ref[...],
                            preferred_element_type=jnp.float32)
                    + bkj_ref[...]) * rh
        v = jnp.dot(xkj.astype(jnp.bfloat16), sc_ref[...],
                    preferred_element_type=jnp.float32)
        for s in range(SCN):
            z_ref[s] = v[:, s * H:(s + 1) * H]

    grid = (E // BE,)
    return pl.pallas_call(
        body,
        grid=grid,
        in_specs=[
            pl.BlockSpec((BE, H), lambda i: (i, 0)),
            pl.BlockSpec((BE, 6), lambda i: (i, 0)),
            pl.BlockSpec((6, H), lambda i: (0, 0)),
            pl.BlockSpec((H, H), lambda i: (0, 0)),
            pl.BlockSpec((1, H), lambda i: (0, 0)),
            pl.BlockSpec((H, H), lambda i: (0, 0)),
            pl.BlockSpec((1, H), lambda i: (0, 0)),
            pl.BlockSpec((H, SCN * H), lambda i: (0, 0)),
        ],
        out_specs=[
            pl.BlockSpec((BE, H), lambda i: (i, 0)),
            pl.BlockSpec((SCN, BE, H), lambda i: (0, i, 0)),
        ],
        out_shape=[
            jax.ShapeDtypeStruct((E, H), jnp.float32),
            jax.ShapeDtypeStruct((SCN, E, H), jnp.float32),
        ],
    )(x, rbf, W_rbf, W_ji, b_ji.reshape(1, H), W_kj, b_kj.reshape(1, H),
      S_cat)


GB = 128
G_STEPS = T // NW // GB


def _sc_gather(table, idx, angle, soff, ts):
    mesh = plsc.VectorSubcoreMesh(core_axis_name="c", subcore_axis_name="s")

    GW = ts // NW

    @functools.partial(
        pl.kernel,
        out_type=jax.ShapeDtypeStruct((ts, H), jnp.float32),
        mesh=mesh,
        scratch_types=[
            pltpu.VMEM((GW,), jnp.int32),
            pltpu.VMEM((GW,), jnp.float32),
            pltpu.VMEM((GW,), jnp.int32),
            pltpu.VMEM((GB, H), jnp.float32),
            pltpu.VMEM((GB, H), jnp.float32),
            pltpu.SemaphoreType.DMA,
            pltpu.SemaphoreType.DMA,
        ],
        compiler_params=pltpu.CompilerParams(needs_layout_passes=False),
    )
    def k(table_hbm, idx_hbm, ang_hbm, out_hbm, idx_v, ang_v, cidx_v,
          rows0, rows1, sem0, sem1):
        wid = lax.axis_index("c") * NS + lax.axis_index("s")
        base = wid * GW

        pltpu.sync_copy(idx_hbm.at[pl.ds(soff + base, GW)], idx_v)
        pltpu.sync_copy(ang_hbm.at[pl.ds(soff + base, GW)], ang_v)

        def cvt(p, _):
            av = ang_v[pl.ds(p * 16, 16)]
            sel = (av / 3.141593 * SCN).astype(jnp.int32)
            iv = idx_v[pl.ds(p * 16, 16)]
            cidx_v[pl.ds(p * 16, 16)] = sel * E + iv
            return 0

        lax.fori_loop(0, GW // 16, cvt, 0, unroll=4)

        def body(i, _):
            g0 = i * 2 * GB
            g1 = g0 + GB
            h0 = pltpu.async_copy(
                table_hbm.at[cidx_v.at[pl.ds(g0, GB)]], rows0, sem0)
            h1 = pltpu.async_copy(
                table_hbm.at[cidx_v.at[pl.ds(g1, GB)]], rows1, sem1)
            h0.wait()
            pltpu.sync_copy(rows0, out_hbm.at[pl.ds(base + g0, GB)])
            h1.wait()
            pltpu.sync_copy(rows1, out_hbm.at[pl.ds(base + g1, GB)])
            return 0

        lax.fori_loop(0, GW // (2 * GB), body, 0, unroll=False)

    return k(table, idx, angle)


BT = 1024


def _triplet(t, sbf_t, W_sbf, B_cat, soff, ts):
    sblk = soff // BT
    def body(t_ref, sbf_ref, wsbf_ref, bcat_ref, y_ref):
        c = lax.dot_general(sbf_ref[...], wsbf_ref[...],
                            (((0,), (0,)), ((), ())),
                            preferred_element_type=jnp.float32)
        tb = t_ref[...]
        sc = jnp.concatenate(
            [(tb * c[:, j][:, None]).astype(jnp.bfloat16)
             for j in range(SCN)], axis=1)
        y_ref[...] = jnp.dot(sc, bcat_ref[...],
                             preferred_element_type=jnp.float32)

    grid = (ts // BT,)
    return pl.pallas_call(
        body,
        grid=grid,
        in_specs=[
            pl.BlockSpec((BT, H), lambda i: (i, 0)),
            pl.BlockSpec((42, BT), lambda i: (0, i + sblk)),
            pl.BlockSpec((42, SCN), lambda i: (0, 0)),
            pl.BlockSpec((SCN * H, H), lambda i: (0, 0)),
        ],
        out_specs=pl.BlockSpec((BT, H), lambda i: (i, 0)),
        out_shape=jax.ShapeDtypeStruct((ts, H), jnp.float32),
    )(t, sbf_t, W_sbf, B_cat)


CR = 8192
NCHUNK = E // CR
CPC = NCHUNK // NC
TRASH = CR
ACC_ROWS = CR + 64
FL = 96


def _sc_scatter_add(y, idx, soff, ts):
    TPW = ts // NS
    CAP = TPW + 2 * FL + 16
    mesh = plsc.VectorSubcoreMesh(core_axis_name="c", subcore_axis_name="s")

    @functools.partial(
        pl.kernel,
        out_type=jax.ShapeDtypeStruct((E, H), jnp.float32),
        mesh=mesh,
        scratch_types=[
            pltpu.VMEM((TPW,), jnp.int32),
            pltpu.VMEM((CAP,), jnp.int32),
            pltpu.VMEM((FL,), jnp.int32),
            pltpu.VMEM((FL,), jnp.int32),
            pltpu.VMEM((FL,), jnp.int32),
            pltpu.VMEM((FL,), jnp.int32),
            pltpu.VMEM((FL, H), jnp.float32),
            pltpu.VMEM((FL, H), jnp.float32),
            pltpu.VMEM_SHARED((ACC_ROWS, H), jnp.float32),
            pltpu.SemaphoreType.DMA,
            pltpu.SemaphoreType.DMA,
        ],
        compiler_params=pltpu.CompilerParams(needs_layout_passes=False),
    )
    def k(y_hbm, idx_hbm, out_hbm, idxs, pkflat, wsm0, dsm0, wsm1,
          dsm1, rows0, rows1, accum, sem0, sem1):
        cid = lax.axis_index("c")
        tid = lax.axis_index("s")
        tbase = tid * TPW

        pltpu.sync_copy(idx_hbm.at[pl.ds(soff + tbase, TPW)], idxs)

        for ck in range(CPC):
            chunk = cid * CPC + ck
            cbase = chunk * CR

            def zb(i, _):
                rows0[i // 8, pl.ds((i % 8) * 16, 16)] = jnp.zeros(
                    (16,), jnp.float32)
                return 0

            lax.fori_loop(0, FL * (H // 16), zb, 0, unroll=False)
            for j in range(5):
                pltpu.sync_copy(
                    rows0, accum.at[pl.ds(tid * 516 + j * FL, FL)])
            pltpu.sync_copy(rows0.at[pl.ds(0, 36)],
                            accum.at[pl.ds(tid * 516 + 5 * FL, 36)])
            plsc.subcore_barrier()

            def inner(kk, off):
                iv = idxs[pl.ds(kk * 16, 16)]
                m = (iv >> 13) == chunk
                w = tbase + kk * 16 + lax.iota(jnp.int32, 16)
                dloc = iv & (CR - 1)
                pk = (w << 14) | dloc
                _, pks = plsc.sort_key_val(1 - m.astype(jnp.int32), pk)
                pkflat[pl.ds(off, 16)] = pks
                return off + jnp.sum(m.astype(jnp.int32))

            off = lax.fori_loop(0, TPW // 16, inner, jnp.int32(0),
                                unroll=4)

            def pad(p, _):
                pkflat[pl.ds(off + p * 16, 16)] = jnp.full(
                    (16,), TRASH + tid, jnp.int32)
                return 0

            lax.fori_loop(0, 2 * FL // 16, pad, 0, unroll=False)


            nf2 = (off + 2 * FL - 1) // (2 * FL)

            def prep(f, wsm, dsm):
                def cp(p, _):
                    pk = pkflat[pl.ds(f * FL + p * 16, 16)]
                    wsm[pl.ds(p * 16, 16)] = lax.shift_right_logical(pk, 14)
                    dsm[pl.ds(p * 16, 16)] = pk & 16383
                    return 0

                lax.fori_loop(0, FL // 16, cp, 0, unroll=6)

            def flush(f2, _):
                prep(2 * f2, wsm0, dsm0)
                h0 = pltpu.async_copy(y_hbm.at[wsm0], rows0, sem0)
                prep(2 * f2 + 1, wsm1, dsm1)
                h1 = pltpu.async_copy(y_hbm.at[wsm1], rows1, sem1)
                h0.wait()
                pltpu.sync_copy(rows0, accum.at[dsm0], add=True)
                h1.wait()
                pltpu.sync_copy(rows1, accum.at[dsm1], add=True)
                return 0

            lax.fori_loop(0, nf2, flush, 0, unroll=False)
            plsc.subcore_barrier()

            pltpu.sync_copy(accum.at[pl.ds(tid * (CR // NS), CR // NS)],
                            out_hbm.at[pl.ds(cbase + tid * (CR // NS),
                                             CR // NS)])
            plsc.subcore_barrier()

    return k(y, idx)


def _final_mlp(x, x_ji, acc, W_bs1, b_bs1, W_bs2, b_bs2, W_lin, b_lin,
               W_as1a, b_as1a, W_as1b, b_as1b, W_as2a, b_as2a, W_as2b,
               b_as2b):
    def body(x_ref, xji_ref, a0_ref, w1_ref, c1_ref, w2_ref, c2_ref,
             wl_ref, cl_ref, wa_ref, ca_ref, wb_ref, cb_ref, wc_ref, cc_ref,
             wd_ref, cd_ref, out_ref):
        def mm(v, w_ref, b_ref):
            return jnp.dot(v, w_ref[...],
                           preferred_element_type=jnp.float32) + b_ref[...]

        h = xji_ref[...] + a0_ref[...]
        h = h + _silu(mm(_silu(mm(h, w1_ref, c1_ref)), w2_ref, c2_ref))
        h = _silu(mm(h, wl_ref, cl_ref)) + x_ref[...]
        h = h + _silu(mm(_silu(mm(h, wa_ref, ca_ref)), wb_ref, cb_ref))
        h = h + _silu(mm(_silu(mm(h, wc_ref, cc_ref)), wd_ref, cd_ref))
        out_ref[...] = h

    grid = (E // BE,)
    row = pl.BlockSpec((BE, H), lambda i: (i, 0))
    wspec = pl.BlockSpec((H, H), lambda i: (0, 0))
    bspec = pl.BlockSpec((1, H), lambda i: (0, 0))
    return pl.pallas_call(
        body,
        grid=grid,
        in_specs=[row, row, row] + [wspec, bspec] * 7,
        out_specs=row,
        out_shape=jax.ShapeDtypeStruct((E, H), jnp.float32),
    )(x, x_ji, acc, W_bs1, b_bs1.reshape(1, H), W_bs2, b_bs2.reshape(1, H),
      W_lin, b_lin.reshape(1, H), W_as1a, b_as1a.reshape(1, H),
      W_as1b, b_as1b.reshape(1, H), W_as2a, b_as2a.reshape(1, H),
      W_as2b, b_as2b.reshape(1, H))


def kernel(x, rbf, sbf, idx_kj, idx_ji, angle, W_rbf, W_sbf, W_ji, b_ji,
           W_kj, b_kj, sel_W, W_bil, W_bs1, b_bs1, W_bs2, b_bs2, W_lin,
           b_lin, W_as1a, b_as1a, W_as1b, b_as1b, W_as2a, b_as2a, W_as2b,
           b_as2b):
    S_cat = jnp.transpose(sel_W, (1, 0, 2)).reshape(H, SCN * H)
    x_ji, z = _edge_prep(x, rbf, W_rbf, W_ji, b_ji, W_kj, b_kj,
                         S_cat.astype(jnp.bfloat16))
    zt = z.reshape(SCN * E, H)
    B_cat = jnp.transpose(W_bil, (1, 2, 0)).reshape(SCN * H, H)
    B_cat16 = B_cat.astype(jnp.bfloat16)
    sbf_t = sbf.T
    t = _sc_gather(zt, idx_kj, angle, 0, T)
    y = _triplet(t, sbf_t, W_sbf, B_cat16, 0, T)
    acc = _sc_scatter_add(y, idx_ji, 0, T)
    return _final_mlp(x, x_ji, acc, W_bs1, b_bs1, W_bs2, b_bs2,
                      W_lin, b_lin, W_as1a, b_as1a, W_as1b, b_as1b, W_as2a,
                      b_as2a, W_as2b, b_as2b)


# speedup vs baseline: 1.0470x; 1.0470x over previous
import functools

import jax
import jax.numpy as jnp
from jax import lax
from jax.experimental import pallas as pl
from jax.experimental.pallas import tpu as pltpu
from jax.experimental.pallas import tpu_sc as plsc

H = 128
E = 65536
T = 262144
SCN = 8

NC = 2
NS = 16
NW = NC * NS


def _silu(v):
    return v / (1.0 + jnp.exp(-v))


BE = 2048


def _edge_prep(x, rbf, W_rbf, W_ji, b_ji, W_kj, b_kj, S_cat):
    def body(x_ref, rbf_ref, wr_ref, wji_ref, bji_ref, wkj_ref, bkj_ref,
             sc_ref, xji_ref, z_ref):
        xb = x_ref[...]
        rh = jnp.dot(rbf_ref[...], wr_ref[...],
                     preferred_element_type=jnp.float32)
        xji_ref[...] = _silu(jnp.dot(xb, wji_ref[...],
                                     preferred_element_type=jnp.float32)
                             + bji_ref[...])
        xkj = _silu(jnp.dot(xb, wkj_ref[...],
                            preferred_element_type=jnp.float32)
                    + bkj_ref[...]) * rh
        v = jnp.dot(xkj.astype(jnp.bfloat16), sc_ref[...],
                    preferred_element_type=jnp.float32)
        for s in range(SCN):
            z_ref[s] = v[:, s * H:(s + 1) * H]

    grid = (E // BE,)
    return pl.pallas_call(
        body,
        grid=grid,
        in_specs=[
            pl.BlockSpec((BE, H), lambda i: (i, 0)),
            pl.BlockSpec((BE, 6), lambda i: (i, 0)),
            pl.BlockSpec((6, H), lambda i: (0, 0)),
            pl.BlockSpec((H, H), lambda i: (0, 0)),
            pl.BlockSpec((1, H), lambda i: (0, 0)),
            pl.BlockSpec((H, H), lambda i: (0, 0)),
            pl.BlockSpec((1, H), lambda i: (0, 0)),
            pl.BlockSpec((H, SCN * H), lambda i: (0, 0)),
        ],
        out_specs=[
            pl.BlockSpec((BE, H), lambda i: (i, 0)),
            pl.BlockSpec((SCN, BE, H), lambda i: (0, i, 0)),
        ],
        out_shape=[
            jax.ShapeDtypeStruct((E, H), jnp.float32),
            jax.ShapeDtypeStruct((SCN, E, H), jnp.float32),
        ],
    )(x, rbf, W_rbf, W_ji, b_ji.reshape(1, H), W_kj, b_kj.reshape(1, H),
      S_cat)


GB = 128
G_STEPS = T // NW // GB


def _sc_gather(table, idx, angle, soff, ts):
    mesh = plsc.VectorSubcoreMesh(core_axis_name="c", subcore_axis_name="s")

    GW = ts // NW

    @functools.partial(
        pl.kernel,
        out_type=jax.ShapeDtypeStruct((ts, H), jnp.float32),
        mesh=mesh,
        scratch_types=[
            pltpu.VMEM((GW,), jnp.int32),
            pltpu.VMEM((GW,), jnp.float32),
            pltpu.VMEM((GW,), jnp.int32),
            pltpu.VMEM((GB, H), jnp.float32),
            pltpu.VMEM((GB, H), jnp.float32),
            pltpu.SemaphoreType.DMA,
            pltpu.SemaphoreType.DMA,
        ],
        compiler_params=pltpu.CompilerParams(needs_layout_passes=False),
    )
    def k(table_hbm, idx_hbm, ang_hbm, out_hbm, idx_v, ang_v, cidx_v,
          rows0, rows1, sem0, sem1):
        wid = lax.axis_index("c") * NS + lax.axis_index("s")
        base = wid * GW

        pltpu.sync_copy(idx_hbm.at[pl.ds(soff + base, GW)], idx_v)
        pltpu.sync_copy(ang_hbm.at[pl.ds(soff + base, GW)], ang_v)

        def cvt(p, _):
            av = ang_v[pl.ds(p * 16, 16)]
            sel = (av / 3.141593 * SCN).astype(jnp.int32)
            iv = idx_v[pl.ds(p * 16, 16)]
            cidx_v[pl.ds(p * 16, 16)] = sel * E + iv
            return 0

        lax.fori_loop(0, GW // 16, cvt, 0, unroll=4)

        def body(i, _):
            g0 = i * 2 * GB
            g1 = g0 + GB
            h0 = pltpu.async_copy(
                table_hbm.at[cidx_v.at[pl.ds(g0, GB)]], rows0, sem0)
            h1 = pltpu.async_copy(
                table_hbm.at[cidx_v.at[pl.ds(g1, GB)]], rows1, sem1)
            h0.wait()
            pltpu.sync_copy(rows0, out_hbm.at[pl.ds(base + g0, GB)])
            h1.wait()
            pltpu.sync_copy(rows1, out_hbm.at[pl.ds(base + g1, GB)])
            return 0

        lax.fori_loop(0, GW // (2 * GB), body, 0, unroll=False)

    return k(table, idx, angle)


BT = 1024


def _triplet(t, sbf_t, W_sbf, B_cat, soff, ts):
    sblk = soff // BT
    def body(t_ref, sbf_ref, wsbf_ref, bcat_ref, y_ref):
        c = lax.dot_general(sbf_ref[...], wsbf_ref[...],
                            (((0,), (0,)), ((), ())),
                            preferred_element_type=jnp.float32)
        v = jnp.dot(t_ref[...].astype(jnp.bfloat16), bcat_ref[...],
                    preferred_element_type=jnp.float32)
        y = jnp.zeros((BT, H), jnp.float32)
        for j in range(SCN):
            y = y + c[:, j][:, None] * v[:, j * H:(j + 1) * H]
        y_ref[...] = y

    grid = (ts // BT,)
    return pl.pallas_call(
        body,
        grid=grid,
        in_specs=[
            pl.BlockSpec((BT, H), lambda i: (i, 0)),
            pl.BlockSpec((42, BT), lambda i: (0, i + sblk)),
            pl.BlockSpec((42, SCN), lambda i: (0, 0)),
            pl.BlockSpec((H, SCN * H), lambda i: (0, 0)),
        ],
        out_specs=pl.BlockSpec((BT, H), lambda i: (i, 0)),
        out_shape=jax.ShapeDtypeStruct((ts, H), jnp.float32),
    )(t, sbf_t, W_sbf, B_cat)


CR = 8192
NCHUNK = E // CR
CPC = NCHUNK // NC
TRASH = CR
ACC_ROWS = CR + 64
FL = 96


def _sc_scatter_add(y, idx, soff, ts):
    TPW = ts // NS
    CAP = TPW + 2 * FL + 16
    mesh = plsc.VectorSubcoreMesh(core_axis_name="c", subcore_axis_name="s")

    @functools.partial(
        pl.kernel,
        out_type=jax.ShapeDtypeStruct((E, H), jnp.float32),
        mesh=mesh,
        scratch_types=[
            pltpu.VMEM((TPW,), jnp.int32),
            pltpu.VMEM((CAP,), jnp.int32),
            pltpu.VMEM((FL,), jnp.int32),
            pltpu.VMEM((FL,), jnp.int32),
            pltpu.VMEM((FL,), jnp.int32),
            pltpu.VMEM((FL,), jnp.int32),
            pltpu.VMEM((FL, H), jnp.float32),
            pltpu.VMEM((FL, H), jnp.float32),
            pltpu.VMEM_SHARED((ACC_ROWS, H), jnp.float32),
            pltpu.SemaphoreType.DMA,
            pltpu.SemaphoreType.DMA,
        ],
        compiler_params=pltpu.CompilerParams(needs_layout_passes=False),
    )
    def k(y_hbm, idx_hbm, out_hbm, idxs, pkflat, wsm0, dsm0, wsm1,
          dsm1, rows0, rows1, accum, sem0, sem1):
        cid = lax.axis_index("c")
        tid = lax.axis_index("s")
        tbase = tid * TPW

        pltpu.sync_copy(idx_hbm.at[pl.ds(soff + tbase, TPW)], idxs)

        for ck in range(CPC):
            chunk = cid * CPC + ck
            cbase = chunk * CR

            def zb(i, _):
                rows0[i // 8, pl.ds((i % 8) * 16, 16)] = jnp.zeros(
                    (16,), jnp.float32)
                return 0

            lax.fori_loop(0, FL * (H // 16), zb, 0, unroll=False)
            for j in range(5):
                pltpu.sync_copy(
                    rows0, accum.at[pl.ds(tid * 516 + j * FL, FL)])
            pltpu.sync_copy(rows0.at[pl.ds(0, 36)],
                            accum.at[pl.ds(tid * 516 + 5 * FL, 36)])
            plsc.subcore_barrier()

            def inner(kk, off):
                iv = idxs[pl.ds(kk * 16, 16)]
                m = (iv >> 13) == chunk
                w = tbase + kk * 16 + lax.iota(jnp.int32, 16)
                dloc = iv & (CR - 1)
                pk = (w << 14) | dloc
                _, pks = plsc.sort_key_val(1 - m.astype(jnp.int32), pk)
                pkflat[pl.ds(off, 16)] = pks
                return off + jnp.sum(m.astype(jnp.int32))

            off = lax.fori_loop(0, TPW // 16, inner, jnp.int32(0),
                                unroll=4)

            def pad(p, _):
                pkflat[pl.ds(off + p * 16, 16)] = jnp.full(
                    (16,), TRASH + tid, jnp.int32)
                return 0

            lax.fori_loop(0, 2 * FL // 16, pad, 0, unroll=False)


            nf2 = (off + 2 * FL - 1) // (2 * FL)

            def prep(f, wsm, dsm):
                def cp(p, _):
                    pk = pkflat[pl.ds(f * FL + p * 16, 16)]
                    wsm[pl.ds(p * 16, 16)] = lax.shift_right_logical(pk, 14)
                    dsm[pl.ds(p * 16, 16)] = pk & 16383
                    return 0

                lax.fori_loop(0, FL // 16, cp, 0, unroll=6)

            def flush(f2, _):
                prep(2 * f2, wsm0, dsm0)
                h0 = pltpu.async_copy(y_hbm.at[wsm0], rows0, sem0)
                prep(2 * f2 + 1, wsm1, dsm1)
                h1 = pltpu.async_copy(y_hbm.at[wsm1], rows1, sem1)
                h0.wait()
                pltpu.sync_copy(rows0, accum.at[dsm0], add=True)
                h1.wait()
                pltpu.sync_copy(rows1, accum.at[dsm1], add=True)
                return 0

            lax.fori_loop(0, nf2, flush, 0, unroll=False)
            plsc.subcore_barrier()

            pltpu.sync_copy(accum.at[pl.ds(tid * (CR // NS), CR // NS)],
                            out_hbm.at[pl.ds(cbase + tid * (CR // NS),
                                             CR // NS)])
            plsc.subcore_barrier()

    return k(y, idx)


def _final_mlp(x, x_ji, acc, W_bs1, b_bs1, W_bs2, b_bs2, W_lin, b_lin,
               W_as1a, b_as1a, W_as1b, b_as1b, W_as2a, b_as2a, W_as2b,
               b_as2b):
    def body(x_ref, xji_ref, a0_ref, w1_ref, c1_ref, w2_ref, c2_ref,
             wl_ref, cl_ref, wa_ref, ca_ref, wb_ref, cb_ref, wc_ref, cc_ref,
             wd_ref, cd_ref, out_ref):
        def mm(v, w_ref, b_ref):
            return jnp.dot(v, w_ref[...],
                           preferred_element_type=jnp.float32) + b_ref[...]

        h = xji_ref[...] + a0_ref[...]
        h = h + _silu(mm(_silu(mm(h, w1_ref, c1_ref)), w2_ref, c2_ref))
        h = _silu(mm(h, wl_ref, cl_ref)) + x_ref[...]
        h = h + _silu(mm(_silu(mm(h, wa_ref, ca_ref)), wb_ref, cb_ref))
        h = h + _silu(mm(_silu(mm(h, wc_ref, cc_ref)), wd_ref, cd_ref))
        out_ref[...] = h

    grid = (E // BE,)
    row = pl.BlockSpec((BE, H), lambda i: (i, 0))
    wspec = pl.BlockSpec((H, H), lambda i: (0, 0))
    bspec = pl.BlockSpec((1, H), lambda i: (0, 0))
    return pl.pallas_call(
        body,
        grid=grid,
        in_specs=[row, row, row] + [wspec, bspec] * 7,
        out_specs=row,
        out_shape=jax.ShapeDtypeStruct((E, H), jnp.float32),
    )(x, x_ji, acc, W_bs1, b_bs1.reshape(1, H), W_bs2, b_bs2.reshape(1, H),
      W_lin, b_lin.reshape(1, H), W_as1a, b_as1a.reshape(1, H),
      W_as1b, b_as1b.reshape(1, H), W_as2a, b_as2a.reshape(1, H),
      W_as2b, b_as2b.reshape(1, H))


def kernel(x, rbf, sbf, idx_kj, idx_ji, angle, W_rbf, W_sbf, W_ji, b_ji,
           W_kj, b_kj, sel_W, W_bil, W_bs1, b_bs1, W_bs2, b_bs2, W_lin,
           b_lin, W_as1a, b_as1a, W_as1b, b_as1b, W_as2a, b_as2a, W_as2b,
           b_as2b):
    S_cat = jnp.transpose(sel_W, (1, 0, 2)).reshape(H, SCN * H)
    x_ji, z = _edge_prep(x, rbf, W_rbf, W_ji, b_ji, W_kj, b_kj,
                         S_cat.astype(jnp.bfloat16))
    zt = z.reshape(SCN * E, H)
    B_cat = jnp.transpose(W_bil, (2, 1, 0)).reshape(H, SCN * H)
    B_cat16 = B_cat.astype(jnp.bfloat16)
    sbf_t = sbf.T
    t = _sc_gather(zt, idx_kj, angle, 0, T)
    y = _triplet(t, sbf_t, W_sbf, B_cat16, 0, T)
    acc = _sc_scatter_add(y, idx_ji, 0, T)
    return _final_mlp(x, x_ji, acc, W_bs1, b_bs1, W_bs2, b_bs2,
                      W_lin, b_lin, W_as1a, b_as1a, W_as1b, b_as1b, W_as2a,
                      b_as2a, W_as2b, b_as2b)


# speedup vs baseline: 1.0871x; 1.0383x over previous
import functools

import jax
import jax.numpy as jnp
from jax import lax
from jax.experimental import pallas as pl
from jax.experimental.pallas import tpu as pltpu
from jax.experimental.pallas import tpu_sc as plsc

H = 128
E = 65536
T = 262144
SCN = 8

NC = 2
NS = 16
NW = NC * NS


def _silu(v):
    return v / (1.0 + jnp.exp(-v))


BE = 2048


def _edge_prep(x, rbf, W_rbf, W_ji, b_ji, W_kj, b_kj, S_cat):
    def body(x_ref, rbf_ref, wr_ref, wji_ref, bji_ref, wkj_ref, bkj_ref,
             sc_ref, xji_ref, z_ref):
        xb = x_ref[...]
        rh = jnp.dot(rbf_ref[...], wr_ref[...],
                     preferred_element_type=jnp.float32)
        xji_ref[...] = _silu(jnp.dot(xb, wji_ref[...],
                                     preferred_element_type=jnp.float32)
                             + bji_ref[...])
        xkj = _silu(jnp.dot(xb, wkj_ref[...],
                            preferred_element_type=jnp.float32)
                    + bkj_ref[...]) * rh
        v = jnp.dot(xkj.astype(jnp.bfloat16), sc_ref[...],
                    preferred_element_type=jnp.float32)
        for s in range(SCN):
            z_ref[s] = v[:, s * H:(s + 1) * H]

    grid = (E // BE,)
    return pl.pallas_call(
        body,
        grid=grid,
        in_specs=[
            pl.BlockSpec((BE, H), lambda i: (i, 0)),
            pl.BlockSpec((BE, 6), lambda i: (i, 0)),
            pl.BlockSpec((6, H), lambda i: (0, 0)),
            pl.BlockSpec((H, H), lambda i: (0, 0)),
            pl.BlockSpec((1, H), lambda i: (0, 0)),
            pl.BlockSpec((H, H), lambda i: (0, 0)),
            pl.BlockSpec((1, H), lambda i: (0, 0)),
            pl.BlockSpec((H, SCN * H), lambda i: (0, 0)),
        ],
        out_specs=[
            pl.BlockSpec((BE, H), lambda i: (i, 0)),
            pl.BlockSpec((SCN, BE, H), lambda i: (0, i, 0)),
        ],
        out_shape=[
            jax.ShapeDtypeStruct((E, H), jnp.float32),
            jax.ShapeDtypeStruct((SCN, E, H), jnp.float32),
        ],
    )(x, rbf, W_rbf, W_ji, b_ji.reshape(1, H), W_kj, b_kj.reshape(1, H),
      S_cat)


GB = 128
G_STEPS = T // NW // GB


def _sc_gather(table, idx, angle, soff, ts):
    mesh = plsc.VectorSubcoreMesh(core_axis_name="c", subcore_axis_name="s")

    GW = ts // NW

    @functools.partial(
        pl.kernel,
        out_type=jax.ShapeDtypeStruct((ts, H), jnp.float32),
        mesh=mesh,
        scratch_types=[
            pltpu.VMEM((GW,), jnp.int32),
            pltpu.VMEM((GW,), jnp.float32),
            pltpu.VMEM((GW,), jnp.int32),
            pltpu.VMEM((GB, H), jnp.float32),
            pltpu.VMEM((GB, H), jnp.float32),
            pltpu.SemaphoreType.DMA,
            pltpu.SemaphoreType.DMA,
        ],
        compiler_params=pltpu.CompilerParams(needs_layout_passes=False),
    )
    def k(table_hbm, idx_hbm, ang_hbm, out_hbm, idx_v, ang_v, cidx_v,
          rows0, rows1, sem0, sem1):
        wid = lax.axis_index("c") * NS + lax.axis_index("s")
        base = wid * GW

        pltpu.sync_copy(idx_hbm.at[pl.ds(soff + base, GW)], idx_v)
        pltpu.sync_copy(ang_hbm.at[pl.ds(soff + base, GW)], ang_v)

        def cvt(p, _):
            av = ang_v[pl.ds(p * 16, 16)]
            sel = (av / 3.141593 * SCN).astype(jnp.int32)
            iv = idx_v[pl.ds(p * 16, 16)]
            cidx_v[pl.ds(p * 16, 16)] = sel * E + iv
            return 0

        lax.fori_loop(0, GW // 16, cvt, 0, unroll=4)

        def body(i, _):
            g0 = i * 2 * GB
            g1 = g0 + GB
            h0 = pltpu.async_copy(
                table_hbm.at[cidx_v.at[pl.ds(g0, GB)]], rows0, sem0)
            h1 = pltpu.async_copy(
                table_hbm.at[cidx_v.at[pl.ds(g1, GB)]], rows1, sem1)
            h0.wait()
            pltpu.sync_copy(rows0, out_hbm.at[pl.ds(base + g0, GB)])
            h1.wait()
            pltpu.sync_copy(rows1, out_hbm.at[pl.ds(base + g1, GB)])
            return 0

        lax.fori_loop(0, GW // (2 * GB), body, 0, unroll=False)

    return k(table, idx, angle)


BT = 2048


def _triplet(t, sbf_t, W_sbf, B_cat, soff, ts):
    sblk = soff // BT
    def body(t_ref, sbf_ref, wsbf_ref, bcat_ref, y_ref):
        c = lax.dot_general(sbf_ref[...], wsbf_ref[...],
                            (((0,), (0,)), ((), ())),
                            preferred_element_type=jnp.float32)
        v = jnp.dot(t_ref[...].astype(jnp.bfloat16), bcat_ref[...],
                    preferred_element_type=jnp.float32)
        y = jnp.zeros((BT, H), jnp.float32)
        for j in range(SCN):
            y = y + c[:, j][:, None] * v[:, j * H:(j + 1) * H]
        y_ref[...] = y

    grid = (ts // BT,)
    return pl.pallas_call(
        body,
        grid=grid,
        in_specs=[
            pl.BlockSpec((BT, H), lambda i: (i, 0)),
            pl.BlockSpec((42, BT), lambda i: (0, i + sblk)),
            pl.BlockSpec((42, SCN), lambda i: (0, 0)),
            pl.BlockSpec((H, SCN * H), lambda i: (0, 0)),
        ],
        out_specs=pl.BlockSpec((BT, H), lambda i: (i, 0)),
        out_shape=jax.ShapeDtypeStruct((ts, H), jnp.float32),
    )(t, sbf_t, W_sbf, B_cat)


CR = 8192
NCHUNK = E // CR
CPC = NCHUNK // NC
TRASH = CR
ACC_ROWS = CR + 64
FL = 96


def _sc_scatter_add(y, idx, soff, ts):
    TPW = ts // NS
    CAP = TPW + 2 * FL + 16
    mesh = plsc.VectorSubcoreMesh(core_axis_name="c", subcore_axis_name="s")

    @functools.partial(
        pl.kernel,
        out_type=jax.ShapeDtypeStruct((E, H), jnp.float32),
        mesh=mesh,
        scratch_types=[
            pltpu.VMEM((TPW,), jnp.int32),
            pltpu.VMEM((CAP,), jnp.int32),
            pltpu.VMEM((FL,), jnp.int32),
            pltpu.VMEM((FL,), jnp.int32),
            pltpu.VMEM((FL,), jnp.int32),
            pltpu.VMEM((FL,), jnp.int32),
            pltpu.VMEM((FL, H), jnp.float32),
            pltpu.VMEM((FL, H), jnp.float32),
            pltpu.VMEM_SHARED((ACC_ROWS, H), jnp.float32),
            pltpu.SemaphoreType.DMA,
            pltpu.SemaphoreType.DMA,
        ],
        compiler_params=pltpu.CompilerParams(needs_layout_passes=False),
    )
    def k(y_hbm, idx_hbm, out_hbm, idxs, pkflat, wsm0, dsm0, wsm1,
          dsm1, rows0, rows1, accum, sem0, sem1):
        cid = lax.axis_index("c")
        tid = lax.axis_index("s")
        tbase = tid * TPW

        pltpu.sync_copy(idx_hbm.at[pl.ds(soff + tbase, TPW)], idxs)

        for ck in range(CPC):
            chunk = cid * CPC + ck
            cbase = chunk * CR

            def zb(i, _):
                rows0[i // 8, pl.ds((i % 8) * 16, 16)] = jnp.zeros(
                    (16,), jnp.float32)
                return 0

            lax.fori_loop(0, FL * (H // 16), zb, 0, unroll=False)
            for j in range(5):
                pltpu.sync_copy(
                    rows0, accum.at[pl.ds(tid * 516 + j * FL, FL)])
            pltpu.sync_copy(rows0.at[pl.ds(0, 36)],
                            accum.at[pl.ds(tid * 516 + 5 * FL, 36)])
            plsc.subcore_barrier()

            def inner(kk, off):
                iv = idxs[pl.ds(kk * 16, 16)]
                m = (iv >> 13) == chunk
                w = tbase + kk * 16 + lax.iota(jnp.int32, 16)
                dloc = iv & (CR - 1)
                pk = (w << 14) | dloc
                _, pks = plsc.sort_key_val(1 - m.astype(jnp.int32), pk)
                pkflat[pl.ds(off, 16)] = pks
                return off + jnp.sum(m.astype(jnp.int32))

            off = lax.fori_loop(0, TPW // 16, inner, jnp.int32(0),
                                unroll=4)

            def pad(p, _):
                pkflat[pl.ds(off + p * 16, 16)] = jnp.full(
                    (16,), TRASH + tid, jnp.int32)
                return 0

            lax.fori_loop(0, 2 * FL // 16, pad, 0, unroll=False)


            nf2 = (off + 2 * FL - 1) // (2 * FL)

            def prep(f, wsm, dsm):
                def cp(p, _):
                    pk = pkflat[pl.ds(f * FL + p * 16, 16)]
                    wsm[pl.ds(p * 16, 16)] = lax.shift_right_logical(pk, 14)
                    dsm[pl.ds(p * 16, 16)] = pk & 16383
                    return 0

                lax.fori_loop(0, FL // 16, cp, 0, unroll=6)

            def flush(f2, _):
                prep(2 * f2, wsm0, dsm0)
                h0 = pltpu.async_copy(y_hbm.at[wsm0], rows0, sem0)
                prep(2 * f2 + 1, wsm1, dsm1)
                h1 = pltpu.async_copy(y_hbm.at[wsm1], rows1, sem1)
                h0.wait()
                pltpu.sync_copy(rows0, accum.at[dsm0], add=True)
                h1.wait()
                pltpu.sync_copy(rows1, accum.at[dsm1], add=True)
                return 0

            lax.fori_loop(0, nf2, flush, 0, unroll=False)
            plsc.subcore_barrier()

            pltpu.sync_copy(accum.at[pl.ds(tid * (CR // NS), CR // NS)],
                            out_hbm.at[pl.ds(cbase + tid * (CR // NS),
                                             CR // NS)])
            plsc.subcore_barrier()

    return k(y, idx)


def _final_mlp(x, x_ji, acc, W_bs1, b_bs1, W_bs2, b_bs2, W_lin, b_lin,
               W_as1a, b_as1a, W_as1b, b_as1b, W_as2a, b_as2a, W_as2b,
               b_as2b):
    def body(x_ref, xji_ref, a0_ref, w1_ref, c1_ref, w2_ref, c2_ref,
             wl_ref, cl_ref, wa_ref, ca_ref, wb_ref, cb_ref, wc_ref, cc_ref,
             wd_ref, cd_ref, out_ref):
        def mm(v, w_ref, b_ref):
            return jnp.dot(v, w_ref[...],
                           preferred_element_type=jnp.float32) + b_ref[...]

        h = xji_ref[...] + a0_ref[...]
        h = h + _silu(mm(_silu(mm(h, w1_ref, c1_ref)), w2_ref, c2_ref))
        h = _silu(mm(h, wl_ref, cl_ref)) + x_ref[...]
        h = h + _silu(mm(_silu(mm(h, wa_ref, ca_ref)), wb_ref, cb_ref))
        h = h + _silu(mm(_silu(mm(h, wc_ref, cc_ref)), wd_ref, cd_ref))
        out_ref[...] = h

    grid = (E // BE,)
    row = pl.BlockSpec((BE, H), lambda i: (i, 0))
    wspec = pl.BlockSpec((H, H), lambda i: (0, 0))
    bspec = pl.BlockSpec((1, H), lambda i: (0, 0))
    return pl.pallas_call(
        body,
        grid=grid,
        in_specs=[row, row, row] + [wspec, bspec] * 7,
        out_specs=row,
        out_shape=jax.ShapeDtypeStruct((E, H), jnp.float32),
    )(x, x_ji, acc, W_bs1, b_bs1.reshape(1, H), W_bs2, b_bs2.reshape(1, H),
      W_lin, b_lin.reshape(1, H), W_as1a, b_as1a.reshape(1, H),
      W_as1b, b_as1b.reshape(1, H), W_as2a, b_as2a.reshape(1, H),
      W_as2b, b_as2b.reshape(1, H))


def kernel(x, rbf, sbf, idx_kj, idx_ji, angle, W_rbf, W_sbf, W_ji, b_ji,
           W_kj, b_kj, sel_W, W_bil, W_bs1, b_bs1, W_bs2, b_bs2, W_lin,
           b_lin, W_as1a, b_as1a, W_as1b, b_as1b, W_as2a, b_as2a, W_as2b,
           b_as2b):
    S_cat = jnp.transpose(sel_W, (1, 0, 2)).reshape(H, SCN * H)
    x_ji, z = _edge_prep(x, rbf, W_rbf, W_ji, b_ji, W_kj, b_kj,
                         S_cat.astype(jnp.bfloat16))
    zt = z.reshape(SCN * E, H)
    B_cat = jnp.transpose(W_bil, (2, 1, 0)).reshape(H, SCN * H)
    B_cat16 = B_cat.astype(jnp.bfloat16)
    sbf_t = sbf.T
    t = _sc_gather(zt, idx_kj, angle, 0, T)
    y = _triplet(t, sbf_t, W_sbf, B_cat16, 0, T)
    acc = _sc_scatter_add(y, idx_ji, 0, T)
    return _final_mlp(x, x_ji, acc, W_bs1, b_bs1, W_bs2, b_bs2,
                      W_lin, b_lin, W_as1a, b_as1a, W_as1b, b_as1b, W_as2a,
                      b_as2a, W_as2b, b_as2b)


# speedup vs baseline: 1.0942x; 1.0065x over previous
import functools

import jax
import jax.numpy as jnp
from jax import lax
from jax.experimental import pallas as pl
from jax.experimental.pallas import tpu as pltpu
from jax.experimental.pallas import tpu_sc as plsc

H = 128
E = 65536
T = 262144
SCN = 8

NC = 2
NS = 16
NW = NC * NS


def _silu(v):
    return v / (1.0 + jnp.exp(-v))


BE = 2048


def _edge_prep(x, rbf, W_rbf, W_ji, b_ji, W_kj, b_kj, S_cat):
    def body(x_ref, rbf_ref, wr_ref, wji_ref, bji_ref, wkj_ref, bkj_ref,
             sc_ref, xji_ref, z_ref):
        xb = x_ref[...]
        rh = jnp.dot(rbf_ref[...], wr_ref[...],
                     preferred_element_type=jnp.float32)
        xji_ref[...] = _silu(jnp.dot(xb, wji_ref[...],
                                     preferred_element_type=jnp.float32)
                             + bji_ref[...])
        xkj = _silu(jnp.dot(xb, wkj_ref[...],
                            preferred_element_type=jnp.float32)
                    + bkj_ref[...]) * rh
        v = jnp.dot(xkj.astype(jnp.bfloat16), sc_ref[...],
                    preferred_element_type=jnp.float32)
        for s in range(SCN):
            z_ref[s] = v[:, s * H:(s + 1) * H]

    grid = (E // BE,)
    return pl.pallas_call(
        body,
        grid=grid,
        in_specs=[
            pl.BlockSpec((BE, H), lambda i: (i, 0)),
            pl.BlockSpec((BE, 6), lambda i: (i, 0)),
            pl.BlockSpec((6, H), lambda i: (0, 0)),
            pl.BlockSpec((H, H), lambda i: (0, 0)),
            pl.BlockSpec((1, H), lambda i: (0, 0)),
            pl.BlockSpec((H, H), lambda i: (0, 0)),
            pl.BlockSpec((1, H), lambda i: (0, 0)),
            pl.BlockSpec((H, SCN * H), lambda i: (0, 0)),
        ],
        out_specs=[
            pl.BlockSpec((BE, H), lambda i: (i, 0)),
            pl.BlockSpec((SCN, BE, H), lambda i: (0, i, 0)),
        ],
        out_shape=[
            jax.ShapeDtypeStruct((E, H), jnp.float32),
            jax.ShapeDtypeStruct((SCN, E, H), jnp.float32),
        ],
    )(x, rbf, W_rbf, W_ji, b_ji.reshape(1, H), W_kj, b_kj.reshape(1, H),
      S_cat)


GB = 128
G_STEPS = T // NW // GB


def _sc_gather(table, idx, angle, soff, ts):
    mesh = plsc.VectorSubcoreMesh(core_axis_name="c", subcore_axis_name="s")

    GW = ts // NW

    @functools.partial(
        pl.kernel,
        out_type=jax.ShapeDtypeStruct((ts, H), jnp.float32),
        mesh=mesh,
        scratch_types=[
            pltpu.VMEM((GW,), jnp.int32),
            pltpu.VMEM((GW,), jnp.float32),
            pltpu.VMEM((GW,), jnp.int32),
            pltpu.VMEM((GB, H), jnp.float32),
            pltpu.VMEM((GB, H), jnp.float32),
            pltpu.SemaphoreType.DMA,
            pltpu.SemaphoreType.DMA,
        ],
        compiler_params=pltpu.CompilerParams(needs_layout_passes=False),
    )
    def k(table_hbm, idx_hbm, ang_hbm, out_hbm, idx_v, ang_v, cidx_v,
          rows0, rows1, sem0, sem1):
        wid = lax.axis_index("c") * NS + lax.axis_index("s")
        base = wid * GW

        pltpu.sync_copy(idx_hbm.at[pl.ds(soff + base, GW)], idx_v)
        pltpu.sync_copy(ang_hbm.at[pl.ds(soff + base, GW)], ang_v)

        def cvt(p, _):
            av = ang_v[pl.ds(p * 16, 16)]
            sel = (av / 3.141593 * SCN).astype(jnp.int32)
            iv = idx_v[pl.ds(p * 16, 16)]
            cidx_v[pl.ds(p * 16, 16)] = sel * E + iv
            return 0

        lax.fori_loop(0, GW // 16, cvt, 0, unroll=4)

        def body(i, _):
            g0 = i * 2 * GB
            g1 = g0 + GB
            h0 = pltpu.async_copy(
                table_hbm.at[cidx_v.at[pl.ds(g0, GB)]], rows0, sem0)
            h1 = pltpu.async_copy(
                table_hbm.at[cidx_v.at[pl.ds(g1, GB)]], rows1, sem1)
            h0.wait()
            pltpu.sync_copy(rows0, out_hbm.at[pl.ds(base + g0, GB)])
            h1.wait()
            pltpu.sync_copy(rows1, out_hbm.at[pl.ds(base + g1, GB)])
            return 0

        lax.fori_loop(0, GW // (2 * GB), body, 0, unroll=False)

    return k(table, idx, angle)


BT = 4096


def _triplet(t, sbf_t, W_sbf, B_cat, soff, ts):
    sblk = soff // BT
    def body(t_ref, sbf_ref, wsbf_ref, bcat_ref, y_ref):
        c = lax.dot_general(sbf_ref[...], wsbf_ref[...],
                            (((0,), (0,)), ((), ())),
                            preferred_element_type=jnp.float32)
        v = jnp.dot(t_ref[...].astype(jnp.bfloat16), bcat_ref[...],
                    preferred_element_type=jnp.float32)
        y = jnp.zeros((BT, H), jnp.float32)
        for j in range(SCN):
            y = y + c[:, j][:, None] * v[:, j * H:(j + 1) * H]
        y_ref[...] = y

    grid = (ts // BT,)
    return pl.pallas_call(
        body,
        grid=grid,
        in_specs=[
            pl.BlockSpec((BT, H), lambda i: (i, 0)),
            pl.BlockSpec((42, BT), lambda i: (0, i + sblk)),
            pl.BlockSpec((42, SCN), lambda i: (0, 0)),
            pl.BlockSpec((H, SCN * H), lambda i: (0, 0)),
        ],
        out_specs=pl.BlockSpec((BT, H), lambda i: (i, 0)),
        out_shape=jax.ShapeDtypeStruct((ts, H), jnp.float32),
    )(t, sbf_t, W_sbf, B_cat)


CR = 8192
NCHUNK = E // CR
CPC = NCHUNK // NC
TRASH = CR
ACC_ROWS = CR + 64
FL = 96


def _sc_scatter_add(y, idx, soff, ts):
    TPW = ts // NS
    CAP = TPW + 2 * FL + 16
    mesh = plsc.VectorSubcoreMesh(core_axis_name="c", subcore_axis_name="s")

    @functools.partial(
        pl.kernel,
        out_type=jax.ShapeDtypeStruct((E, H), jnp.float32),
        mesh=mesh,
        scratch_types=[
            pltpu.VMEM((TPW,), jnp.int32),
            pltpu.VMEM((CAP,), jnp.int32),
            pltpu.VMEM((FL,), jnp.int32),
            pltpu.VMEM((FL,), jnp.int32),
            pltpu.VMEM((FL,), jnp.int32),
            pltpu.VMEM((FL,), jnp.int32),
            pltpu.VMEM((FL, H), jnp.float32),
            pltpu.VMEM((FL, H), jnp.float32),
            pltpu.VMEM_SHARED((ACC_ROWS, H), jnp.float32),
            pltpu.SemaphoreType.DMA,
            pltpu.SemaphoreType.DMA,
        ],
        compiler_params=pltpu.CompilerParams(needs_layout_passes=False),
    )
    def k(y_hbm, idx_hbm, out_hbm, idxs, pkflat, wsm0, dsm0, wsm1,
          dsm1, rows0, rows1, accum, sem0, sem1):
        cid = lax.axis_index("c")
        tid = lax.axis_index("s")
        tbase = tid * TPW

        pltpu.sync_copy(idx_hbm.at[pl.ds(soff + tbase, TPW)], idxs)

        for ck in range(CPC):
            chunk = cid * CPC + ck
            cbase = chunk * CR

            def zb(i, _):
                rows0[i // 8, pl.ds((i % 8) * 16, 16)] = jnp.zeros(
                    (16,), jnp.float32)
                return 0

            lax.fori_loop(0, FL * (H // 16), zb, 0, unroll=False)
            for j in range(5):
                pltpu.sync_copy(
                    rows0, accum.at[pl.ds(tid * 516 + j * FL, FL)])
            pltpu.sync_copy(rows0.at[pl.ds(0, 36)],
                            accum.at[pl.ds(tid * 516 + 5 * FL, 36)])
            plsc.subcore_barrier()

            def inner(kk, off):
                iv = idxs[pl.ds(kk * 16, 16)]
                m = (iv >> 13) == chunk
                w = tbase + kk * 16 + lax.iota(jnp.int32, 16)
                dloc = iv & (CR - 1)
                pk = (w << 14) | dloc
                _, pks = plsc.sort_key_val(1 - m.astype(jnp.int32), pk)
                pkflat[pl.ds(off, 16)] = pks
                return off + jnp.sum(m.astype(jnp.int32))

            off = lax.fori_loop(0, TPW // 16, inner, jnp.int32(0),
                                unroll=4)

            def pad(p, _):
                pkflat[pl.ds(off + p * 16, 16)] = jnp.full(
                    (16,), TRASH + tid, jnp.int32)
                return 0

            lax.fori_loop(0, 2 * FL // 16, pad, 0, unroll=False)


            nf2 = (off + 2 * FL - 1) // (2 * FL)

            def prep(f, wsm, dsm):
                def cp(p, _):
                    pk = pkflat[pl.ds(f * FL + p * 16, 16)]
                    wsm[pl.ds(p * 16, 16)] = lax.shift_right_logical(pk, 14)
                    dsm[pl.ds(p * 16, 16)] = pk & 16383
                    return 0

                lax.fori_loop(0, FL // 16, cp, 0, unroll=6)

            def flush(f2, _):
                prep(2 * f2, wsm0, dsm0)
                h0 = pltpu.async_copy(y_hbm.at[wsm0], rows0, sem0)
                prep(2 * f2 + 1, wsm1, dsm1)
                h1 = pltpu.async_copy(y_hbm.at[wsm1], rows1, sem1)
                h0.wait()
                pltpu.sync_copy(rows0, accum.at[dsm0], add=True)
                h1.wait()
                pltpu.sync_copy(rows1, accum.at[dsm1], add=True)
                return 0

            lax.fori_loop(0, nf2, flush, 0, unroll=False)
            plsc.subcore_barrier()

            pltpu.sync_copy(accum.at[pl.ds(tid * (CR // NS), CR // NS)],
                            out_hbm.at[pl.ds(cbase + tid * (CR // NS),
                                             CR // NS)])
            plsc.subcore_barrier()

    return k(y, idx)


def _final_mlp(x, x_ji, acc, W_bs1, b_bs1, W_bs2, b_bs2, W_lin, b_lin,
               W_as1a, b_as1a, W_as1b, b_as1b, W_as2a, b_as2a, W_as2b,
               b_as2b):
    def body(x_ref, xji_ref, a0_ref, w1_ref, c1_ref, w2_ref, c2_ref,
             wl_ref, cl_ref, wa_ref, ca_ref, wb_ref, cb_ref, wc_ref, cc_ref,
             wd_ref, cd_ref, out_ref):
        def mm(v, w_ref, b_ref):
            return jnp.dot(v, w_ref[...],
                           preferred_element_type=jnp.float32) + b_ref[...]

        h = xji_ref[...] + a0_ref[...]
        h = h + _silu(mm(_silu(mm(h, w1_ref, c1_ref)), w2_ref, c2_ref))
        h = _silu(mm(h, wl_ref, cl_ref)) + x_ref[...]
        h = h + _silu(mm(_silu(mm(h, wa_ref, ca_ref)), wb_ref, cb_ref))
        h = h + _silu(mm(_silu(mm(h, wc_ref, cc_ref)), wd_ref, cd_ref))
        out_ref[...] = h

    grid = (E // BE,)
    row = pl.BlockSpec((BE, H), lambda i: (i, 0))
    wspec = pl.BlockSpec((H, H), lambda i: (0, 0))
    bspec = pl.BlockSpec((1, H), lambda i: (0, 0))
    return pl.pallas_call(
        body,
        grid=grid,
        in_specs=[row, row, row] + [wspec, bspec] * 7,
        out_specs=row,
        out_shape=jax.ShapeDtypeStruct((E, H), jnp.float32),
    )(x, x_ji, acc, W_bs1, b_bs1.reshape(1, H), W_bs2, b_bs2.reshape(1, H),
      W_lin, b_lin.reshape(1, H), W_as1a, b_as1a.reshape(1, H),
      W_as1b, b_as1b.reshape(1, H), W_as2a, b_as2a.reshape(1, H),
      W_as2b, b_as2b.reshape(1, H))


def kernel(x, rbf, sbf, idx_kj, idx_ji, angle, W_rbf, W_sbf, W_ji, b_ji,
           W_kj, b_kj, sel_W, W_bil, W_bs1, b_bs1, W_bs2, b_bs2, W_lin,
           b_lin, W_as1a, b_as1a, W_as1b, b_as1b, W_as2a, b_as2a, W_as2b,
           b_as2b):
    S_cat = jnp.transpose(sel_W, (1, 0, 2)).reshape(H, SCN * H)
    x_ji, z = _edge_prep(x, rbf, W_rbf, W_ji, b_ji, W_kj, b_kj,
                         S_cat.astype(jnp.bfloat16))
    zt = z.reshape(SCN * E, H)
    B_cat = jnp.transpose(W_bil, (2, 1, 0)).reshape(H, SCN * H)
    B_cat16 = B_cat.astype(jnp.bfloat16)
    sbf_t = sbf.T
    t = _sc_gather(zt, idx_kj, angle, 0, T)
    y = _triplet(t, sbf_t, W_sbf, B_cat16, 0, T)
    acc = _sc_scatter_add(y, idx_ji, 0, T)
    return _final_mlp(x, x_ji, acc, W_bs1, b_bs1, W_bs2, b_bs2,
                      W_lin, b_lin, W_as1a, b_as1a, W_as1b, b_as1b, W_as2a,
                      b_as2a, W_as2b, b_as2b)


# speedup vs baseline: 1.1009x; 1.0061x over previous
import functools

import jax
import jax.numpy as jnp
from jax import lax
from jax.experimental import pallas as pl
from jax.experimental.pallas import tpu as pltpu
from jax.experimental.pallas import tpu_sc as plsc

H = 128
E = 65536
T = 262144
SCN = 8

NC = 2
NS = 16
NW = NC * NS


def _silu(v):
    return v / (1.0 + jnp.exp(-v))


BE = 4096


def _edge_prep(x, rbf, W_rbf, W_ji, b_ji, W_kj, b_kj, S_cat):
    def body(x_ref, rbf_ref, wr_ref, wji_ref, bji_ref, wkj_ref, bkj_ref,
             sc_ref, xji_ref, z_ref):
        xb = x_ref[...]
        rh = jnp.dot(rbf_ref[...], wr_ref[...],
                     preferred_element_type=jnp.float32)
        xji_ref[...] = _silu(jnp.dot(xb, wji_ref[...],
                                     preferred_element_type=jnp.float32)
                             + bji_ref[...])
        xkj = _silu(jnp.dot(xb, wkj_ref[...],
                            preferred_element_type=jnp.float32)
                    + bkj_ref[...]) * rh
        v = jnp.dot(xkj.astype(jnp.bfloat16), sc_ref[...],
                    preferred_element_type=jnp.float32)
        for s in range(SCN):
            z_ref[s] = v[:, s * H:(s + 1) * H]

    grid = (E // BE,)
    return pl.pallas_call(
        body,
        grid=grid,
        in_specs=[
            pl.BlockSpec((BE, H), lambda i: (i, 0)),
            pl.BlockSpec((BE, 6), lambda i: (i, 0)),
            pl.BlockSpec((6, H), lambda i: (0, 0)),
            pl.BlockSpec((H, H), lambda i: (0, 0)),
            pl.BlockSpec((1, H), lambda i: (0, 0)),
            pl.BlockSpec((H, H), lambda i: (0, 0)),
            pl.BlockSpec((1, H), lambda i: (0, 0)),
            pl.BlockSpec((H, SCN * H), lambda i: (0, 0)),
        ],
        out_specs=[
            pl.BlockSpec((BE, H), lambda i: (i, 0)),
            pl.BlockSpec((SCN, BE, H), lambda i: (0, i, 0)),
        ],
        out_shape=[
            jax.ShapeDtypeStruct((E, H), jnp.float32),
            jax.ShapeDtypeStruct((SCN, E, H), jnp.float32),
        ],
    )(x, rbf, W_rbf, W_ji, b_ji.reshape(1, H), W_kj, b_kj.reshape(1, H),
      S_cat)


GB = 128
G_STEPS = T // NW // GB


def _sc_gather(table, idx, angle, soff, ts):
    mesh = plsc.VectorSubcoreMesh(core_axis_name="c", subcore_axis_name="s")

    GW = ts // NW

    @functools.partial(
        pl.kernel,
        out_type=jax.ShapeDtypeStruct((ts, H), jnp.float32),
        mesh=mesh,
        scratch_types=[
            pltpu.VMEM((GW,), jnp.int32),
            pltpu.VMEM((GW,), jnp.float32),
            pltpu.VMEM((GW,), jnp.int32),
            pltpu.VMEM((GB, H), jnp.float32),
            pltpu.VMEM((GB, H), jnp.float32),
            pltpu.SemaphoreType.DMA,
            pltpu.SemaphoreType.DMA,
        ],
        compiler_params=pltpu.CompilerParams(needs_layout_passes=False),
    )
    def k(table_hbm, idx_hbm, ang_hbm, out_hbm, idx_v, ang_v, cidx_v,
          rows0, rows1, sem0, sem1):
        wid = lax.axis_index("c") * NS + lax.axis_index("s")
        base = wid * GW

        pltpu.sync_copy(idx_hbm.at[pl.ds(soff + base, GW)], idx_v)
        pltpu.sync_copy(ang_hbm.at[pl.ds(soff + base, GW)], ang_v)

        def cvt(p, _):
            av = ang_v[pl.ds(p * 16, 16)]
            sel = (av / 3.141593 * SCN).astype(jnp.int32)
            iv = idx_v[pl.ds(p * 16, 16)]
            cidx_v[pl.ds(p * 16, 16)] = sel * E + iv
            return 0

        lax.fori_loop(0, GW // 16, cvt, 0, unroll=4)

        def body(i, _):
            g0 = i * 2 * GB
            g1 = g0 + GB
            h0 = pltpu.async_copy(
                table_hbm.at[cidx_v.at[pl.ds(g0, GB)]], rows0, sem0)
            h1 = pltpu.async_copy(
                table_hbm.at[cidx_v.at[pl.ds(g1, GB)]], rows1, sem1)
            h0.wait()
            pltpu.sync_copy(rows0, out_hbm.at[pl.ds(base + g0, GB)])
            h1.wait()
            pltpu.sync_copy(rows1, out_hbm.at[pl.ds(base + g1, GB)])
            return 0

        lax.fori_loop(0, GW // (2 * GB), body, 0, unroll=False)

    return k(table, idx, angle)


BT = 4096


def _triplet(t, sbf_t, W_sbf, B_cat, soff, ts):
    sblk = soff // BT
    def body(t_ref, sbf_ref, wsbf_ref, bcat_ref, y_ref):
        c = lax.dot_general(sbf_ref[...], wsbf_ref[...],
                            (((0,), (0,)), ((), ())),
                            preferred_element_type=jnp.float32)
        v = jnp.dot(t_ref[...].astype(jnp.bfloat16), bcat_ref[...],
                    preferred_element_type=jnp.float32)
        y = jnp.zeros((BT, H), jnp.float32)
        for j in range(SCN):
            y = y + c[:, j][:, None] * v[:, j * H:(j + 1) * H]
        y_ref[...] = y

    grid = (ts // BT,)
    return pl.pallas_call(
        body,
        grid=grid,
        in_specs=[
            pl.BlockSpec((BT, H), lambda i: (i, 0)),
            pl.BlockSpec((42, BT), lambda i: (0, i + sblk)),
            pl.BlockSpec((42, SCN), lambda i: (0, 0)),
            pl.BlockSpec((H, SCN * H), lambda i: (0, 0)),
        ],
        out_specs=pl.BlockSpec((BT, H), lambda i: (i, 0)),
        out_shape=jax.ShapeDtypeStruct((ts, H), jnp.float32),
    )(t, sbf_t, W_sbf, B_cat)


CR = 8192
NCHUNK = E // CR
CPC = NCHUNK // NC
TRASH = CR
ACC_ROWS = CR + 64
FL = 96


def _sc_scatter_add(y, idx, soff, ts):
    TPW = ts // NS
    CAP = TPW + 2 * FL + 16
    mesh = plsc.VectorSubcoreMesh(core_axis_name="c", subcore_axis_name="s")

    @functools.partial(
        pl.kernel,
        out_type=jax.ShapeDtypeStruct((E, H), jnp.float32),
        mesh=mesh,
        scratch_types=[
            pltpu.VMEM((TPW,), jnp.int32),
            pltpu.VMEM((CAP,), jnp.int32),
            pltpu.VMEM((FL,), jnp.int32),
            pltpu.VMEM((FL,), jnp.int32),
            pltpu.VMEM((FL,), jnp.int32),
            pltpu.VMEM((FL,), jnp.int32),
            pltpu.VMEM((FL, H), jnp.float32),
            pltpu.VMEM((FL, H), jnp.float32),
            pltpu.VMEM_SHARED((ACC_ROWS, H), jnp.float32),
            pltpu.SemaphoreType.DMA,
            pltpu.SemaphoreType.DMA,
        ],
        compiler_params=pltpu.CompilerParams(needs_layout_passes=False),
    )
    def k(y_hbm, idx_hbm, out_hbm, idxs, pkflat, wsm0, dsm0, wsm1,
          dsm1, rows0, rows1, accum, sem0, sem1):
        cid = lax.axis_index("c")
        tid = lax.axis_index("s")
        tbase = tid * TPW

        pltpu.sync_copy(idx_hbm.at[pl.ds(soff + tbase, TPW)], idxs)

        for ck in range(CPC):
            chunk = cid * CPC + ck
            cbase = chunk * CR

            def zb(i, _):
                rows0[i // 8, pl.ds((i % 8) * 16, 16)] = jnp.zeros(
                    (16,), jnp.float32)
                return 0

            lax.fori_loop(0, FL * (H // 16), zb, 0, unroll=False)
            for j in range(5):
                pltpu.sync_copy(
                    rows0, accum.at[pl.ds(tid * 516 + j * FL, FL)])
            pltpu.sync_copy(rows0.at[pl.ds(0, 36)],
                            accum.at[pl.ds(tid * 516 + 5 * FL, 36)])
            plsc.subcore_barrier()

            def inner(kk, off):
                iv = idxs[pl.ds(kk * 16, 16)]
                m = (iv >> 13) == chunk
                w = tbase + kk * 16 + lax.iota(jnp.int32, 16)
                dloc = iv & (CR - 1)
                pk = (w << 14) | dloc
                _, pks = plsc.sort_key_val(1 - m.astype(jnp.int32), pk)
                pkflat[pl.ds(off, 16)] = pks
                return off + jnp.sum(m.astype(jnp.int32))

            off = lax.fori_loop(0, TPW // 16, inner, jnp.int32(0),
                                unroll=4)

            def pad(p, _):
                pkflat[pl.ds(off + p * 16, 16)] = jnp.full(
                    (16,), TRASH + tid, jnp.int32)
                return 0

            lax.fori_loop(0, 2 * FL // 16, pad, 0, unroll=False)


            nf2 = (off + 2 * FL - 1) // (2 * FL)

            def prep(f, wsm, dsm):
                def cp(p, _):
                    pk = pkflat[pl.ds(f * FL + p * 16, 16)]
                    wsm[pl.ds(p * 16, 16)] = lax.shift_right_logical(pk, 14)
                    dsm[pl.ds(p * 16, 16)] = pk & 16383
                    return 0

                lax.fori_loop(0, FL // 16, cp, 0, unroll=6)

            def flush(f2, _):
                prep(2 * f2, wsm0, dsm0)
                h0 = pltpu.async_copy(y_hbm.at[wsm0], rows0, sem0)
                prep(2 * f2 + 1, wsm1, dsm1)
                h1 = pltpu.async_copy(y_hbm.at[wsm1], rows1, sem1)
                h0.wait()
                pltpu.sync_copy(rows0, accum.at[dsm0], add=True)
                h1.wait()
                pltpu.sync_copy(rows1, accum.at[dsm1], add=True)
                return 0

            lax.fori_loop(0, nf2, flush, 0, unroll=False)
            plsc.subcore_barrier()

            pltpu.sync_copy(accum.at[pl.ds(tid * (CR // NS), CR // NS)],
                            out_hbm.at[pl.ds(cbase + tid * (CR // NS),
                                             CR // NS)])
            plsc.subcore_barrier()

    return k(y, idx)


def _final_mlp(x, x_ji, acc, W_bs1, b_bs1, W_bs2, b_bs2, W_lin, b_lin,
               W_as1a, b_as1a, W_as1b, b_as1b, W_as2a, b_as2a, W_as2b,
               b_as2b):
    def body(x_ref, xji_ref, a0_ref, w1_ref, c1_ref, w2_ref, c2_ref,
             wl_ref, cl_ref, wa_ref, ca_ref, wb_ref, cb_ref, wc_ref, cc_ref,
             wd_ref, cd_ref, out_ref):
        def mm(v, w_ref, b_ref):
            return jnp.dot(v, w_ref[...],
                           preferred_element_type=jnp.float32) + b_ref[...]

        h = xji_ref[...] + a0_ref[...]
        h = h + _silu(mm(_silu(mm(h, w1_ref, c1_ref)), w2_ref, c2_ref))
        h = _silu(mm(h, wl_ref, cl_ref)) + x_ref[...]
        h = h + _silu(mm(_silu(mm(h, wa_ref, ca_ref)), wb_ref, cb_ref))
        h = h + _silu(mm(_silu(mm(h, wc_ref, cc_ref)), wd_ref, cd_ref))
        out_ref[...] = h

    grid = (E // BE,)
    row = pl.BlockSpec((BE, H), lambda i: (i, 0))
    wspec = pl.BlockSpec((H, H), lambda i: (0, 0))
    bspec = pl.BlockSpec((1, H), lambda i: (0, 0))
    return pl.pallas_call(
        body,
        grid=grid,
        in_specs=[row, row, row] + [wspec, bspec] * 7,
        out_specs=row,
        out_shape=jax.ShapeDtypeStruct((E, H), jnp.float32),
    )(x, x_ji, acc, W_bs1, b_bs1.reshape(1, H), W_bs2, b_bs2.reshape(1, H),
      W_lin, b_lin.reshape(1, H), W_as1a, b_as1a.reshape(1, H),
      W_as1b, b_as1b.reshape(1, H), W_as2a, b_as2a.reshape(1, H),
      W_as2b, b_as2b.reshape(1, H))


def kernel(x, rbf, sbf, idx_kj, idx_ji, angle, W_rbf, W_sbf, W_ji, b_ji,
           W_kj, b_kj, sel_W, W_bil, W_bs1, b_bs1, W_bs2, b_bs2, W_lin,
           b_lin, W_as1a, b_as1a, W_as1b, b_as1b, W_as2a, b_as2a, W_as2b,
           b_as2b):
    S_cat = jnp.transpose(sel_W, (1, 0, 2)).reshape(H, SCN * H)
    x_ji, z = _edge_prep(x, rbf, W_rbf, W_ji, b_ji, W_kj, b_kj,
                         S_cat.astype(jnp.bfloat16))
    zt = z.reshape(SCN * E, H)
    B_cat = jnp.transpose(W_bil, (2, 1, 0)).reshape(H, SCN * H)
    B_cat16 = B_cat.astype(jnp.bfloat16)
    sbf_t = sbf.T
    t = _sc_gather(zt, idx_kj, angle, 0, T)
    y = _triplet(t, sbf_t, W_sbf, B_cat16, 0, T)
    acc = _sc_scatter_add(y, idx_ji, 0, T)
    return _final_mlp(x, x_ji, acc, W_bs1, b_bs1, W_bs2, b_bs2,
                      W_lin, b_lin, W_as1a, b_as1a, W_as1b, b_as1b, W_as2a,
                      b_as2a, W_as2b, b_as2b)
